# baseline (speedup 1.0000x reference)
import functools

import jax
import jax.numpy as jnp
from jax import lax
from jax.experimental import pallas as pl
from jax.experimental.pallas import tpu as pltpu
from jax.experimental.pallas import tpu_sc as plsc


def _make_sc_kernel(B, H, W, D):
    SUB = (2 * D) // 128
    XS = D // 128
    WB = W // 8
    ROWS = WB * SUB * 8
    mesh = plsc.VectorSubcoreMesh(core_axis_name="c", subcore_axis_name="s")

    @functools.partial(
        pl.kernel,
        mesh=mesh,
        out_type=jax.ShapeDtypeStruct((B * H * ROWS, 128), jnp.float32),
        scratch_types=[
            pltpu.VMEM((4, 128), jnp.int32),
            pltpu.VMEM((4 * ROWS, 128), jnp.float32),
            pltpu.SemaphoreType.DMA,
        ],
    )
    def k(comb_hbm, idx_hbm, out_hbm, idx_v, blk_v, sem):
        wid = lax.axis_index("s") * 2 + lax.axis_index("c")
        qg = wid % 8
        b0 = wid // 8
        h0 = 4 * qg
        pltpu.sync_copy(idx_hbm.at[pl.ds(h0, 4)], idx_v)
        gathers = [
            pltpu.make_async_copy(
                comb_hbm.at[idx_v.at[j]],
                blk_v.at[pl.ds(j * ROWS, ROWS)], sem)
            for j in range(4)
        ]
        for g in gathers:
            g.start()
        for g in gathers:
            g.wait()
        outs = []
        for b in (b0, b0 + 4):
            dst0 = (b * H + h0) * ROWS
            outs.append(pltpu.make_async_copy(
                blk_v, out_hbm.at[pl.ds(dst0, 4 * ROWS)], sem))
        for c in outs:
            c.start()
        for c in outs:
            c.wait()

    return k


def kernel(row_embeddings, column_embeddings, x):
    batch, _, height, width = x.shape
    D = row_embeddings.shape[1]
    C = 2 * D
    XS = D // 128
    colP = column_embeddings[:width].reshape(width * XS, 128)
    rowP = row_embeddings[:height].reshape(height * XS, 128)
    comb = jnp.concatenate([colP, rowP], axis=0)
    idx_tab = []
    for h in range(height):
        idx_h = []
        for j in range(128):
            wb, r = divmod(j, 4 * 8)
            cb, w8 = divmod(r, 8)
            if cb < XS:
                idx_h.append(XS * (8 * wb + w8) + cb)
            else:
                idx_h.append(width * XS + XS * h + cb - XS)
        idx_tab.append(idx_h)
    idx_tab = jnp.asarray(idx_tab, dtype=jnp.int32)
    k = _make_sc_kernel(batch, height, width, D)
    out = k(comb, idx_tab)
    out6 = out.reshape(batch, height, width // 8, C // 128, 8, 128)
    return out6.transpose(0, 3, 5, 1, 2, 4).reshape(batch, C, height, width)


# speedup vs baseline: 6.7128x; 6.7128x over previous
import jax
import jax.numpy as jnp
from jax import lax
from jax.experimental import pallas as pl
from jax.experimental.pallas import tpu as pltpu

_CHUNKS = 4


def _pos_kernel(row_ref, col_ref, out_ref, scratch, sem):
    H, W, D = 32, 32, 256
    HW = H * W
    B = out_ref.shape[0]
    HC = H // _CHUNKS
    R = HC * W
    col = col_ref[0:W, :]
    row = row_ref[0:H, :]
    copies = []
    for g in range(_CHUNKS):
        x_tile = jnp.concatenate([col] * HC, axis=0)
        j = lax.broadcasted_iota(jnp.int32, (R, H), 0) + g * R
        hsel = lax.broadcasted_iota(jnp.int32, (R, H), 1)
        rep = (j // W == hsel).astype(jnp.float32)
        dn = (((1,), (0,)), ((), ()))
        y_tile = lax.dot_general(rep, row, dn,
                                 preferred_element_type=jnp.float32)
        scratch[pl.ds(g * R, R), :] = jnp.concatenate([x_tile, y_tile], axis=1)
        for b in range(B):
            c = pltpu.make_async_copy(
                scratch.at[pl.ds(g * R, R)],
                out_ref.at[b, pl.ds(g * R, R)],
                sem.at[b],
            )
            c.start()
            copies.append(c)
    for c in copies:
        c.wait()


def kernel(row_embeddings, column_embeddings, x):
    batch, _, height, width = x.shape
    D = row_embeddings.shape[1]
    C = 2 * D
    HW = height * width
    out = pl.pallas_call(
        _pos_kernel,
        in_specs=[
            pl.BlockSpec(memory_space=pltpu.MemorySpace.VMEM),
            pl.BlockSpec(memory_space=pltpu.MemorySpace.VMEM),
        ],
        out_specs=pl.BlockSpec(memory_space=pltpu.MemorySpace.HBM),
        out_shape=jax.ShapeDtypeStruct((batch, HW, C), jnp.float32),
        scratch_shapes=[
            pltpu.VMEM((HW, C), jnp.float32),
            pltpu.SemaphoreType.DMA((batch,)),
        ],
    )(row_embeddings, column_embeddings)
    return out.reshape(batch, height, width, C).transpose(0, 3, 1, 2)


# speedup vs baseline: 6.7654x; 1.0078x over previous
import jax
import jax.numpy as jnp
from jax import lax
from jax.experimental import pallas as pl
from jax.experimental.pallas import tpu as pltpu

_CHUNKS = 4


def _pos_kernel(row_hbm, col_hbm, out_ref, tab_v, scratch, sem):
    H, W, D = 32, 32, 256
    HW = H * W
    B = out_ref.shape[0]
    HC = H // _CHUNKS
    R = HC * W
    ld_r = pltpu.make_async_copy(row_hbm, tab_v.at[0], sem.at[0])
    ld_c = pltpu.make_async_copy(col_hbm, tab_v.at[1], sem.at[1])
    ld_r.start()
    ld_c.start()
    ld_r.wait()
    ld_c.wait()
    col = tab_v[1, 0:W, :]
    row = tab_v[0, 0:H, :]
    copies = []
    for g in range(_CHUNKS):
        x_tile = jnp.concatenate([col] * HC, axis=0)
        j = lax.broadcasted_iota(jnp.int32, (R, H), 0) + g * R
        hsel = lax.broadcasted_iota(jnp.int32, (R, H), 1)
        rep = (j // W == hsel).astype(jnp.float32)
        dn = (((1,), (0,)), ((), ()))
        y_tile = lax.dot_general(rep, row, dn,
                                 preferred_element_type=jnp.float32)
        scratch[pl.ds(g * R, R), :] = jnp.concatenate([x_tile, y_tile], axis=1)
        for b in range(B):
            c = pltpu.make_async_copy(
                scratch.at[pl.ds(g * R, R)],
                out_ref.at[b, pl.ds(g * R, R)],
                sem.at[b],
            )
            c.start()
            copies.append(c)
    for c in copies:
        c.wait()


def kernel(row_embeddings, column_embeddings, x):
    batch, _, height, width = x.shape
    D = row_embeddings.shape[1]
    C = 2 * D
    HW = height * width
    out = pl.pallas_call(
        _pos_kernel,
        in_specs=[
            pl.BlockSpec(memory_space=pltpu.MemorySpace.HBM),
            pl.BlockSpec(memory_space=pltpu.MemorySpace.HBM),
        ],
        out_specs=pl.BlockSpec(memory_space=pltpu.MemorySpace.HBM),
        out_shape=jax.ShapeDtypeStruct((batch, HW, C), jnp.float32),
        scratch_shapes=[
            pltpu.VMEM((2,) + row_embeddings.shape, jnp.float32),
            pltpu.VMEM((HW, C), jnp.float32),
            pltpu.SemaphoreType.DMA((batch,)),
        ],
    )(row_embeddings, column_embeddings)
    return out.reshape(batch, height, width, C).transpose(0, 3, 1, 2)


# speedup vs baseline: 6.9891x; 1.0331x over previous
import jax
import jax.numpy as jnp
from jax import lax
from jax.experimental import pallas as pl
from jax.experimental.pallas import tpu as pltpu

_CHUNKS = 4


def _pos_kernel(row_hbm, col_hbm, out_ref, tab_v, scratch, sem):
    H, W, D = 32, 32, 256
    HW = H * W
    B = out_ref.shape[0]
    HC = H // _CHUNKS
    R = HC * W
    ld_r = pltpu.make_async_copy(row_hbm, tab_v.at[0], sem.at[0])
    ld_c = pltpu.make_async_copy(col_hbm, tab_v.at[1], sem.at[1])
    ld_r.start()
    ld_c.start()
    ld_r.wait()
    ld_c.wait()
    col = tab_v[1, 0:W, :]
    row = tab_v[0, 0:H, :]
    copies = []
    for g in range(_CHUNKS):
        x_tile = jnp.concatenate([col] * HC, axis=0)
        rows_g = row[g * HC:(g + 1) * HC, :]
        y_tile = jnp.broadcast_to(
            rows_g[:, None, :], (HC, W, D)).reshape(R, D)
        scratch[pl.ds(g * R, R), :] = jnp.concatenate([x_tile, y_tile], axis=1)
        for b in range(B):
            c = pltpu.make_async_copy(
                scratch.at[pl.ds(g * R, R)],
                out_ref.at[b, pl.ds(g * R, R)],
                sem.at[b],
            )
            c.start()
            copies.append(c)
    for c in copies:
        c.wait()


def kernel(row_embeddings, column_embeddings, x):
    batch, _, height, width = x.shape
    D = row_embeddings.shape[1]
    C = 2 * D
    HW = height * width
    out = pl.pallas_call(
        _pos_kernel,
        in_specs=[
            pl.BlockSpec(memory_space=pltpu.MemorySpace.HBM),
            pl.BlockSpec(memory_space=pltpu.MemorySpace.HBM),
        ],
        out_specs=pl.BlockSpec(memory_space=pltpu.MemorySpace.HBM),
        out_shape=jax.ShapeDtypeStruct((batch, HW, C), jnp.float32),
        scratch_shapes=[
            pltpu.VMEM((2,) + row_embeddings.shape, jnp.float32),
            pltpu.VMEM((HW, C), jnp.float32),
            pltpu.SemaphoreType.DMA((batch,)),
        ],
    )(row_embeddings, column_embeddings)
    return out.reshape(batch, height, width, C).transpose(0, 3, 1, 2)


# speedup vs baseline: 7.0323x; 1.0062x over previous
import jax
import jax.numpy as jnp
from jax import lax
from jax.experimental import pallas as pl
from jax.experimental.pallas import tpu as pltpu

_CHUNKS = 4


def _pos_kernel(row_hbm, col_hbm, out_ref, tab_v, scratch, sem):
    H, W, D = 32, 32, 256
    HW = H * W
    B = out_ref.shape[0]
    HC = H // _CHUNKS
    R = HC * W
    ld_r = pltpu.make_async_copy(row_hbm, tab_v.at[0], sem.at[0])
    ld_c = pltpu.make_async_copy(col_hbm, tab_v.at[1], sem.at[1])
    ld_r.start()
    ld_c.start()
    ld_r.wait()
    ld_c.wait()
    col = tab_v[1, 0:W, :]
    row = tab_v[0, 0:H, :]
    copies = []
    for g in range(_CHUNKS):
        x_tile = jnp.broadcast_to(col[None], (HC, W, D)).reshape(R, D)
        rows_g = row[g * HC:(g + 1) * HC, :]
        y_tile = jnp.broadcast_to(
            rows_g[:, None, :], (HC, W, D)).reshape(R, D)
        scratch[pl.ds(g * R, R), :] = jnp.concatenate([x_tile, y_tile], axis=1)
        for b in range(B):
            c = pltpu.make_async_copy(
                scratch.at[pl.ds(g * R, R)],
                out_ref.at[b, pl.ds(g * R, R)],
                sem.at[b],
            )
            c.start()
            copies.append(c)
    for c in copies:
        c.wait()


def kernel(row_embeddings, column_embeddings, x):
    batch, _, height, width = x.shape
    D = row_embeddings.shape[1]
    C = 2 * D
    HW = height * width
    out = pl.pallas_call(
        _pos_kernel,
        in_specs=[
            pl.BlockSpec(memory_space=pltpu.MemorySpace.HBM),
            pl.BlockSpec(memory_space=pltpu.MemorySpace.HBM),
        ],
        out_specs=pl.BlockSpec(memory_space=pltpu.MemorySpace.HBM),
        out_shape=jax.ShapeDtypeStruct((batch, HW, C), jnp.float32),
        scratch_shapes=[
            pltpu.VMEM((2,) + row_embeddings.shape, jnp.float32),
            pltpu.VMEM((HW, C), jnp.float32),
            pltpu.SemaphoreType.DMA((batch,)),
        ],
    )(row_embeddings, column_embeddings)
    return out.reshape(batch, height, width, C).transpose(0, 3, 1, 2)


# speedup vs baseline: 7.1137x; 1.0116x over previous
import jax
import jax.numpy as jnp
from jax import lax
from jax.experimental import pallas as pl
from jax.experimental.pallas import tpu as pltpu

_CHUNKS = 8


def _pos_kernel(row_hbm, col_hbm, out_ref, tab_v, scratch, sem):
    H, W, D = 32, 32, 256
    HW = H * W
    B = out_ref.shape[0]
    HC = H // _CHUNKS
    R = HC * W
    ld_r = pltpu.make_async_copy(row_hbm, tab_v.at[0], sem.at[0])
    ld_c = pltpu.make_async_copy(col_hbm, tab_v.at[1], sem.at[1])
    ld_r.start()
    ld_c.start()
    ld_r.wait()
    ld_c.wait()
    col = tab_v[1, 0:W, :]
    row = tab_v[0, 0:H, :]
    copies = []
    for g in range(_CHUNKS):
        x_tile = jnp.broadcast_to(col[None], (HC, W, D)).reshape(R, D)
        rows_g = row[g * HC:(g + 1) * HC, :]
        y_tile = jnp.broadcast_to(
            rows_g[:, None, :], (HC, W, D)).reshape(R, D)
        scratch[pl.ds(g * R, R), :] = jnp.concatenate([x_tile, y_tile], axis=1)
        for b in range(B):
            c = pltpu.make_async_copy(
                scratch.at[pl.ds(g * R, R)],
                out_ref.at[b, pl.ds(g * R, R)],
                sem.at[b],
            )
            c.start()
            copies.append(c)
    for c in copies:
        c.wait()


def kernel(row_embeddings, column_embeddings, x):
    batch, _, height, width = x.shape
    D = row_embeddings.shape[1]
    C = 2 * D
    HW = height * width
    out = pl.pallas_call(
        _pos_kernel,
        in_specs=[
            pl.BlockSpec(memory_space=pltpu.MemorySpace.HBM),
            pl.BlockSpec(memory_space=pltpu.MemorySpace.HBM),
        ],
        out_specs=pl.BlockSpec(memory_space=pltpu.MemorySpace.HBM),
        out_shape=jax.ShapeDtypeStruct((batch, HW, C), jnp.float32),
        scratch_shapes=[
            pltpu.VMEM((2,) + row_embeddings.shape, jnp.float32),
            pltpu.VMEM((HW, C), jnp.float32),
            pltpu.SemaphoreType.DMA((batch,)),
        ],
    )(row_embeddings, column_embeddings)
    return out.reshape(batch, height, width, C).transpose(0, 3, 1, 2)


# speedup vs baseline: 7.1335x; 1.0028x over previous
import jax
import jax.numpy as jnp
from jax import lax
from jax.experimental import pallas as pl
from jax.experimental.pallas import tpu as pltpu

_CHUNKS = 16


def _pos_kernel(row_hbm, col_hbm, out_ref, tab_v, scratch, sem):
    H, W, D = 32, 32, 256
    HW = H * W
    B = out_ref.shape[0]
    HC = H // _CHUNKS
    R = HC * W
    ld_r = pltpu.make_async_copy(row_hbm, tab_v.at[0], sem.at[0])
    ld_c = pltpu.make_async_copy(col_hbm, tab_v.at[1], sem.at[1])
    ld_r.start()
    ld_c.start()
    ld_r.wait()
    ld_c.wait()
    col = tab_v[1, 0:W, :]
    row = tab_v[0, 0:H, :]
    copies = []
    for g in range(_CHUNKS):
        x_tile = jnp.broadcast_to(col[None], (HC, W, D)).reshape(R, D)
        rows_g = row[g * HC:(g + 1) * HC, :]
        y_tile = jnp.broadcast_to(
            rows_g[:, None, :], (HC, W, D)).reshape(R, D)
        scratch[pl.ds(g * R, R), :] = jnp.concatenate([x_tile, y_tile], axis=1)
        for b in range(B):
            c = pltpu.make_async_copy(
                scratch.at[pl.ds(g * R, R)],
                out_ref.at[b, pl.ds(g * R, R)],
                sem.at[b],
            )
            c.start()
            copies.append(c)
    for c in copies:
        c.wait()


def kernel(row_embeddings, column_embeddings, x):
    batch, _, height, width = x.shape
    D = row_embeddings.shape[1]
    C = 2 * D
    HW = height * width
    out = pl.pallas_call(
        _pos_kernel,
        in_specs=[
            pl.BlockSpec(memory_space=pltpu.MemorySpace.HBM),
            pl.BlockSpec(memory_space=pltpu.MemorySpace.HBM),
        ],
        out_specs=pl.BlockSpec(memory_space=pltpu.MemorySpace.HBM),
        out_shape=jax.ShapeDtypeStruct((batch, HW, C), jnp.float32),
        scratch_shapes=[
            pltpu.VMEM((2,) + row_embeddings.shape, jnp.float32),
            pltpu.VMEM((HW, C), jnp.float32),
            pltpu.SemaphoreType.DMA((batch,)),
        ],
    )(row_embeddings, column_embeddings)
    return out.reshape(batch, height, width, C).transpose(0, 3, 1, 2)
